# baseline (speedup 1.0000x reference)
import functools

import numpy as np
import jax
import jax.numpy as jnp
from jax import lax
from jax.experimental import pallas as pl
from jax.experimental.pallas import tpu as pltpu
from jax.experimental.pallas import tpu_sc as plsc

_RATIO = 0.75
_NUM_CORES = 2
_NUM_SUBCORES = 16
_NUM_WORKERS = _NUM_CORES * _NUM_SUBCORES
_TILESPMEM_BYTES = 524284

def _rotl(x, d):
    return ((x << np.uint32(d)) | (x >> np.uint32(32 - d))).astype(np.uint32)


def _threefry2x32(k0, k1, x0, x1):
    rot = ((13, 15, 26, 6), (17, 29, 16, 24))
    ks = [np.uint32(k0), np.uint32(k1),
          np.uint32(k0 ^ k1 ^ np.uint32(0x1BD11BDA))]
    x = [x0.astype(np.uint32) + ks[0], x1.astype(np.uint32) + ks[1]]

    def rounds(x, rots):
        for d in rots:
            x[0] = (x[0] + x[1]).astype(np.uint32)
            x[1] = _rotl(x[1], d) ^ x[0]
        return x

    x = rounds(x, rot[0])
    x = [x[0] + ks[1], x[1] + ks[2] + np.uint32(1)]
    x = rounds(x, rot[1])
    x = [x[0] + ks[2], x[1] + ks[0] + np.uint32(2)]
    x = rounds(x, rot[0])
    x = [x[0] + ks[0], x[1] + ks[1] + np.uint32(3)]
    x = rounds(x, rot[1])
    x = [x[0] + ks[1], x[1] + ks[2] + np.uint32(4)]
    x = rounds(x, rot[0])
    return ((x[0] + ks[2]).astype(np.uint32),
            (x[1] + ks[0] + np.uint32(5)).astype(np.uint32))


def _split_keys(k0, k1, num):
    i = np.arange(num, dtype=np.uint32)
    o0, o1 = _threefry2x32(k0, k1, np.zeros(num, np.uint32), i)
    return list(zip(o0, o1))


def _random_bits(k0, k1, n):
    i = np.arange(n, dtype=np.uint32)
    o0, o1 = _threefry2x32(k0, k1, np.zeros(n, np.uint32), i)
    return o0 ^ o1


_idx_cache = {}


def _perm_indexes(B, N):
    key = (B, N)
    if key not in _idx_cache:
        fwd = np.empty((B, N), dtype=np.int32)
        for b, (k0, k1) in enumerate(_split_keys(np.uint32(0), np.uint32(42), B)):
            _, sub = _split_keys(k0, k1, 2)
            fwd[b] = np.argsort(_random_bits(sub[0], sub[1], N),
                                kind="stable").astype(np.int32)
        bwd = np.argsort(fwd, axis=1, kind="stable").astype(np.int32)
        _idx_cache[key] = (fwd, bwd)
    return _idx_cache[key]


def _pick_chunk(rows_per_worker, dim, nbuf):
    limit = min(128, (_TILESPMEM_BYTES - 8192) // (nbuf * dim * 4))
    best = 8
    for c in range(8, limit + 1, 8):
        if rows_per_worker % c == 0:
            best = c
    return best


@functools.lru_cache(maxsize=None)
def _gather_call(rows_pad, dim, chunk, nch, nbuf, idx_elems):
    nch8 = -(-nch // 8) * 8
    seg = idx_elems // _NUM_WORKERS
    mesh = plsc.VectorSubcoreMesh(
        core_axis_name="c", subcore_axis_name="s",
        num_cores=_NUM_CORES, num_subcores=_NUM_SUBCORES)

    @functools.partial(
        pl.kernel,
        out_type=(jax.ShapeDtypeStruct((rows_pad, dim), jnp.float32),
                  jax.ShapeDtypeStruct((idx_elems,), jnp.int32),
                  jax.ShapeDtypeStruct((idx_elems,), jnp.int32)),
        mesh=mesh,
        scratch_types=[
            pltpu.VMEM((nch8, chunk), jnp.int32),
            [pltpu.VMEM((chunk, dim), jnp.float32) for _ in range(nbuf)],
            [pltpu.SemaphoreType.DMA for _ in range(nbuf)],
            [pltpu.SemaphoreType.DMA for _ in range(nbuf)],
            pltpu.SemaphoreType.DMA,
        ],
    )
    def body(table_hbm, idx_hbm, fb_hbm, out_hbm, fwd_hbm, bwd_hbm,
             idx_v, bufs, gsems, wsems, fbsem):
        wid = lax.axis_index("s") * _NUM_CORES + lax.axis_index("c")
        c0 = wid * nch
        fcp = pltpu.async_copy(
            fb_hbm.at[pl.ds(wid * seg, seg)],
            fwd_hbm.at[pl.ds(wid * seg, seg)], fbsem)
        bcp = pltpu.async_copy(
            fb_hbm.at[pl.ds(idx_elems + wid * seg, seg)],
            bwd_hbm.at[pl.ds(wid * seg, seg)], fbsem)
        pltpu.sync_copy(idx_hbm.at[pl.ds(wid * nch8, nch8)], idx_v)
        gcp = [None] * nbuf
        wcp = [None] * nbuf
        depth = min(nbuf - 1, nch)
        for c in range(depth):
            gcp[c % nbuf] = pltpu.async_copy(
                table_hbm.at[idx_v.at[c]], bufs[c % nbuf], gsems[c % nbuf])
        for c in range(nch):
            i = c % nbuf
            j = c + depth
            if j < nch:
                k = j % nbuf
                if wcp[k] is not None:
                    wcp[k].wait()
                gcp[k] = pltpu.async_copy(
                    table_hbm.at[idx_v.at[j]], bufs[k], gsems[k])
            gcp[i].wait()
            wcp[i] = pltpu.async_copy(
                bufs[i], out_hbm.at[pl.ds((c0 + c) * chunk, chunk)], wsems[i])
        for i in range(min(nbuf, nch)):
            if wcp[i] is not None:
                wcp[i].wait()
        fcp.wait()
        bcp.wait()

    return body


def kernel(patches):
    B, N, dim = patches.shape
    remain_N = int(N * (1 - _RATIO))
    fwd, bwd = _perm_indexes(B, N)

    rows = B * remain_N
    flat_idx = (fwd[:, :remain_N]
                + (np.arange(B, dtype=np.int32) * N)[:, None]).reshape(-1)
    rows_pad = -(-rows // (_NUM_WORKERS * 8)) * (_NUM_WORKERS * 8)
    if rows_pad != rows:
        flat_idx = np.concatenate(
            [flat_idx, np.zeros(rows_pad - rows, dtype=np.int32)])
    rpw = rows_pad // _NUM_WORKERS
    nbuf = 3
    chunk = _pick_chunk(rpw, dim, nbuf)
    nch = rpw // chunk
    nch8 = -(-nch // 8) * 8

    table = patches.reshape(B * N, dim)
    idx2d = flat_idx.reshape(_NUM_WORKERS, nch, chunk)
    if nch8 != nch:
        idx2d = np.concatenate(
            [idx2d, np.zeros((_NUM_WORKERS, nch8 - nch, chunk), np.int32)],
            axis=1)
    idx2d = jnp.asarray(idx2d.reshape(_NUM_WORKERS * nch8, chunk))
    fb = jnp.asarray(np.concatenate([fwd.reshape(-1), bwd.reshape(-1)]))
    out, fwd_o, bwd_o = _gather_call(
        rows_pad, dim, chunk, nch, nbuf, B * N)(table, idx2d, fb)
    out = out[:rows].reshape(B, remain_N, dim)
    return (out, fwd_o.reshape(B, N), bwd_o.reshape(B, N))


# speedup vs baseline: 1.0719x; 1.0719x over previous
import functools

import numpy as np
import jax
import jax.numpy as jnp
from jax import lax
from jax.experimental import pallas as pl
from jax.experimental.pallas import tpu as pltpu
from jax.experimental.pallas import tpu_sc as plsc

_RATIO = 0.75
_NUM_CORES = 2
_NUM_SUBCORES = 16
_NUM_WORKERS = _NUM_CORES * _NUM_SUBCORES
_TILESPMEM_BYTES = 524284

def _rotl(x, d):
    return ((x << np.uint32(d)) | (x >> np.uint32(32 - d))).astype(np.uint32)


def _threefry2x32(k0, k1, x0, x1):
    rot = ((13, 15, 26, 6), (17, 29, 16, 24))
    ks = [np.uint32(k0), np.uint32(k1),
          np.uint32(k0 ^ k1 ^ np.uint32(0x1BD11BDA))]
    x = [x0.astype(np.uint32) + ks[0], x1.astype(np.uint32) + ks[1]]

    def rounds(x, rots):
        for d in rots:
            x[0] = (x[0] + x[1]).astype(np.uint32)
            x[1] = _rotl(x[1], d) ^ x[0]
        return x

    x = rounds(x, rot[0])
    x = [x[0] + ks[1], x[1] + ks[2] + np.uint32(1)]
    x = rounds(x, rot[1])
    x = [x[0] + ks[2], x[1] + ks[0] + np.uint32(2)]
    x = rounds(x, rot[0])
    x = [x[0] + ks[0], x[1] + ks[1] + np.uint32(3)]
    x = rounds(x, rot[1])
    x = [x[0] + ks[1], x[1] + ks[2] + np.uint32(4)]
    x = rounds(x, rot[0])
    return ((x[0] + ks[2]).astype(np.uint32),
            (x[1] + ks[0] + np.uint32(5)).astype(np.uint32))


def _split_keys(k0, k1, num):
    i = np.arange(num, dtype=np.uint32)
    o0, o1 = _threefry2x32(k0, k1, np.zeros(num, np.uint32), i)
    return list(zip(o0, o1))


def _random_bits(k0, k1, n):
    i = np.arange(n, dtype=np.uint32)
    o0, o1 = _threefry2x32(k0, k1, np.zeros(n, np.uint32), i)
    return o0 ^ o1


_idx_cache = {}


def _perm_indexes(B, N):
    key = (B, N)
    if key not in _idx_cache:
        fwd = np.empty((B, N), dtype=np.int32)
        for b, (k0, k1) in enumerate(_split_keys(np.uint32(0), np.uint32(42), B)):
            _, sub = _split_keys(k0, k1, 2)
            fwd[b] = np.argsort(_random_bits(sub[0], sub[1], N),
                                kind="stable").astype(np.int32)
        bwd = np.argsort(fwd, axis=1, kind="stable").astype(np.int32)
        _idx_cache[key] = (fwd, bwd)
    return _idx_cache[key]


def _pick_chunk(rows_per_worker, dim, nbuf):
    limit = min(128, (_TILESPMEM_BYTES - 8192) // (nbuf * dim * 4))
    best = 8
    for c in range(8, limit + 1, 8):
        if rows_per_worker % c == 0:
            best = c
    return best


@functools.lru_cache(maxsize=None)
def _gather_call(rows_pad, dim, chunk, nch, nbuf):
    rpw = nch * chunk
    mesh = plsc.VectorSubcoreMesh(
        core_axis_name="c", subcore_axis_name="s",
        num_cores=_NUM_CORES, num_subcores=_NUM_SUBCORES)

    @functools.partial(
        pl.kernel,
        out_type=jax.ShapeDtypeStruct((rows_pad, dim), jnp.float32),
        mesh=mesh,
        scratch_types=[
            pltpu.VMEM((rpw,), jnp.int32),
            [pltpu.VMEM((chunk, dim), jnp.float32) for _ in range(nbuf)],
            [pltpu.SemaphoreType.DMA for _ in range(nbuf)],
            [pltpu.SemaphoreType.DMA for _ in range(nbuf)],
        ],
    )
    def body(table_hbm, idx_hbm, out_hbm, idx_v, bufs, gsems, wsems):
        wid = lax.axis_index("s") * _NUM_CORES + lax.axis_index("c")
        c0 = wid * nch
        pltpu.sync_copy(idx_hbm.at[pl.ds(wid * rpw, rpw)], idx_v)
        gcp = [None] * nbuf
        wcp = [None] * nbuf
        depth = min(nbuf - 1, nch)
        for c in range(depth):
            gcp[c % nbuf] = pltpu.async_copy(
                table_hbm.at[idx_v.at[pl.ds(c * chunk, chunk)]],
                bufs[c % nbuf], gsems[c % nbuf])
        for c in range(nch):
            i = c % nbuf
            j = c + depth
            if j < nch:
                k = j % nbuf
                if wcp[k] is not None:
                    wcp[k].wait()
                gcp[k] = pltpu.async_copy(
                    table_hbm.at[idx_v.at[pl.ds(j * chunk, chunk)]],
                    bufs[k], gsems[k])
            gcp[i].wait()
            wcp[i] = pltpu.async_copy(
                bufs[i], out_hbm.at[pl.ds((c0 + c) * chunk, chunk)], wsems[i])
        for i in range(min(nbuf, nch)):
            if wcp[i] is not None:
                wcp[i].wait()

    return body


def kernel(patches):
    B, N, dim = patches.shape
    remain_N = int(N * (1 - _RATIO))
    fwd, bwd = _perm_indexes(B, N)

    rows = B * remain_N
    flat_idx = (fwd[:, :remain_N]
                + (np.arange(B, dtype=np.int32) * N)[:, None]).reshape(-1)
    rows_pad = -(-rows // (_NUM_WORKERS * 8)) * (_NUM_WORKERS * 8)
    if rows_pad != rows:
        flat_idx = np.concatenate(
            [flat_idx, np.zeros(rows_pad - rows, dtype=np.int32)])
    rpw = rows_pad // _NUM_WORKERS
    nbuf = 3
    chunk = _pick_chunk(rpw, dim, nbuf)
    nch = rpw // chunk

    table = patches.reshape(B * N, dim)
    idx1d = jnp.asarray(flat_idx)
    out = _gather_call(rows_pad, dim, chunk, nch, nbuf)(table, idx1d)
    out = out[:rows].reshape(B, remain_N, dim)
    return (out, jnp.asarray(fwd), jnp.asarray(bwd))
